# baseline (speedup 1.0000x reference)
import functools

import jax
import jax.numpy as jnp
from jax import lax
from jax.experimental import pallas as pl
from jax.experimental.pallas import tpu as pltpu
from jax.experimental.pallas import tpu_sc as plsc

N = 10000
K = 16
_EPS = 1e-5
NPAD = 10240
NCH = NPAD // 128

_SC_NC = 2
_SC_NS = 16
_SC_NW = _SC_NC * _SC_NS


def _sc_gather(table, idx, n_rows, c_out=None):
    T, C = table.shape
    assert C % 128 == 0
    c_out = C if c_out is None else c_out
    n_chunks = n_rows // 128
    cpw = (n_chunks + _SC_NW - 1) // _SC_NW
    cpw = ((cpw + 7) // 8) * 8
    n_pad = cpw * _SC_NW
    lsz = ((cpw + 8 + 7) // 8) * 8
    idx2 = jnp.zeros((n_pad + 32, 128), jnp.int32).at[:n_chunks].set(
        idx.reshape(n_chunks, 128))
    mesh = plsc.VectorSubcoreMesh(core_axis_name="c", subcore_axis_name="s")

    assert cpw % 4 == 0

    @functools.partial(
        pl.kernel,
        mesh=mesh,
        out_type=jax.ShapeDtypeStruct((n_pad * 128, c_out), jnp.float32),
        scratch_types=[
            pltpu.VMEM((lsz, 128), jnp.int32),
            pltpu.VMEM((128, C), jnp.float32),
            pltpu.VMEM((128, C), jnp.float32),
            pltpu.SemaphoreType.DMA,
            pltpu.SemaphoreType.DMA,
        ],
    )
    def gk(idx_hbm, table_hbm, out_hbm, idx_v, rows0, rows1, sem0, sem1):
        wid = lax.axis_index("s") * _SC_NC + lax.axis_index("c")
        base = wid * cpw
        ab = pl.multiple_of((base // 8) * 8, 8)
        off = base - ab
        pltpu.sync_copy(idx_hbm.at[pl.ds(ab, lsz)], idx_v)
        pltpu.async_copy(table_hbm.at[idx_v.at[off]], rows0, sem0)

        def store(rows, j):
            src = rows if c_out == C else rows.at[:, pl.ds(0, c_out)]
            pltpu.sync_copy(src, out_hbm.at[pl.ds((base + j) * 128, 128)])

        def body(h, carry):
            j = 2 * h
            pltpu.async_copy(table_hbm.at[idx_v.at[j + 1]], rows1, sem1)
            pltpu.make_async_copy(table_hbm.at[idx_v.at[j]], rows0, sem0).wait()
            store(rows0, j)

            @pl.when(j + 2 < cpw)
            def _():
                pltpu.async_copy(table_hbm.at[idx_v.at[j + 2]], rows0, sem0)

            pltpu.make_async_copy(table_hbm.at[idx_v.at[j + 1]], rows1, sem1).wait()
            store(rows1, j + 1)
            return carry

        lax.fori_loop(0, cpw // 2, body, 0)

    return gk(idx2, table)[:n_rows]


def _knn_a_body(q_ref, pt_ref, d_ref, seg_ref, ck_ref):
    q = q_ref[...]
    pt = pt_ref[...]
    t = jnp.dot(q, pt, preferred_element_type=jnp.float32)
    sqc = jnp.sum(pt * pt, axis=0, keepdims=True)
    d = sqc - 2.0 * t
    mins = []
    for c in range(NCH):
        blk = d[:, c * 128:(c + 1) * 128]
        d_ref[pl.ds(c * 128, 128), :] = blk
        mins.append(jnp.min(blk, axis=1, keepdims=True))
    m = jnp.concatenate(mins, axis=1)
    iota = lax.broadcasted_iota(jnp.int32, (128, NCH), 1).astype(jnp.float32)
    qloc = lax.broadcasted_iota(jnp.int32, (128, 1), 0)
    cols = []
    for _ in range(K):
        rm = jnp.min(m, axis=1, keepdims=True)
        cand = jnp.where(m == rm, iota, jnp.float32(1e9))
        pc = jnp.min(cand, axis=1, keepdims=True)
        m = jnp.where(cand == pc, jnp.float32(jnp.inf), m)
        cols.append(pc.astype(jnp.int32))
    ck = jnp.concatenate(cols, axis=1)
    ck_ref[...] = ck
    seg_ref[...] = pl.program_id(0) * (NCH * 128) + ck * 128 + qloc


def _knn_stage_a(posq, post):
    return pl.pallas_call(
        _knn_a_body,
        grid=(NPAD // 128,),
        in_specs=[
            pl.BlockSpec((128, 3), lambda i: (i, 0)),
            pl.BlockSpec((3, NPAD), lambda i: (0, 0)),
        ],
        out_specs=[
            pl.BlockSpec((NCH * 128, 128), lambda i: (i, 0)),
            pl.BlockSpec((128, K), lambda i: (i, 0)),
            pl.BlockSpec((128, K), lambda i: (i, 0)),
        ],
        out_shape=[
            jax.ShapeDtypeStruct((NPAD * NCH, 128), jnp.float32),
            jax.ShapeDtypeStruct((NPAD, K), jnp.int32),
            jax.ShapeDtypeStruct((NPAD, K), jnp.int32),
        ],
    )(posq, post)


def _knn_b_body(dr_ref, ck_ref, nbr_ref):
    P = dr_ref.shape[0]
    d = dr_ref[...]
    ck_f = ck_ref[...].astype(jnp.float32)
    iota2 = lax.broadcasted_iota(jnp.int32, (P, K * 128), 1).astype(jnp.float32)
    iota16 = lax.broadcasted_iota(jnp.int32, (P, K), 1).astype(jnp.float32)
    cols = []
    for _ in range(K):
        rm = jnp.min(d, axis=1, keepdims=True)
        cand = jnp.where(d == rm, iota2, jnp.float32(1e9))
        pc = jnp.min(cand, axis=1, keepdims=True)
        d = jnp.where(cand == pc, jnp.float32(jnp.inf), d)
        j2 = jnp.floor(pc * (1.0 / 128.0))
        lane = pc - j2 * 128.0
        sel = jnp.sum(jnp.where(iota16 == j2, ck_f, 0.0), axis=1, keepdims=True)
        cols.append((sel * 128.0 + lane).astype(jnp.int32))
    nbr_ref[...] = jnp.concatenate(cols, axis=1)


def _knn_stage_b(dr, ck, n, P=200):
    return pl.pallas_call(
        _knn_b_body,
        grid=(n // P,),
        in_specs=[
            pl.BlockSpec((P, K * 128), lambda i: (i, 0)),
            pl.BlockSpec((P, K), lambda i: (i, 0)),
        ],
        out_specs=pl.BlockSpec((P, K), lambda i: (i, 0)),
        out_shape=jax.ShapeDtypeStruct((n, K), jnp.int32),
    )(dr, ck)


def _ab_body(x_ref, p_ref, wt_ref, wb_ref, a_ref, b_ref):
    bvec = jnp.dot(p_ref[...], wb_ref[...], preferred_element_type=jnp.float32)
    a_ref[...] = jnp.dot(x_ref[...], wt_ref[...], preferred_element_type=jnp.float32) + bvec
    b_ref[...] = bvec


def _node_ab(x, pos, Wa):
    fin = x.shape[1]
    C = Wa.shape[1]
    wt = Wa[:fin]
    wb = Wa[fin:]
    return pl.pallas_call(
        _ab_body,
        out_shape=(
            jax.ShapeDtypeStruct((N, C), jnp.float32),
            jax.ShapeDtypeStruct((N, C), jnp.float32),
        ),
    )(x, pos, wt, wb)


def _conv_body(g_ref, b_ref, wb_ref, s_ref, prm_ref, o_ref):
    Kc, P, _ = g_ref.shape
    C = b_ref.shape[1]
    ba = prm_ref[0:1, :]
    gm = prm_ref[1:2, :]
    bt = prm_ref[2:3, :]
    bb = prm_ref[3:4, :]
    s = s_ref[...]
    h = (g_ref[..., :C] - b_ref[...][None]).reshape(Kc * P, C) + ba
    m = jnp.dot(h, s, preferred_element_type=jnp.float32)
    d = h - m
    v = jnp.dot(d * d, s, preferred_element_type=jnp.float32)
    hn = (d / jnp.sqrt(v + _EPS)) * gm + bt
    hr = jnp.maximum(hn, 0.0)
    z = jnp.dot(hr, wb_ref[...], preferred_element_type=jnp.float32)
    acc = jnp.max(z.reshape(Kc, P, C), axis=0)
    o_ref[...] = jnp.maximum(acc + bb, 0.0)


def _conv_layer(G, B, Wb, prm, S, P=200):
    n = B.shape[0]
    C = B.shape[1]
    Cg = G.shape[2]
    grid = (n // P,)
    return pl.pallas_call(
        _conv_body,
        grid=grid,
        in_specs=[
            pl.BlockSpec((K, P, Cg), lambda i: (0, i, 0)),
            pl.BlockSpec((P, C), lambda i: (i, 0)),
            pl.BlockSpec((C, C), lambda i: (0, 0)),
            pl.BlockSpec((C, C), lambda i: (0, 0)),
            pl.BlockSpec((8, C), lambda i: (0, 0)),
        ],
        out_specs=pl.BlockSpec((P, C), lambda i: (i, 0)),
        out_shape=jax.ShapeDtypeStruct((n, C), jnp.float32),
    )(G, B, Wb, S, prm)


def _group_avg_matrix(C):
    i = jnp.arange(C)
    return jnp.where((i[:, None] // 8) == (i[None, :] // 8), 1.0 / 8.0, 0.0).astype(jnp.float32)


def _pack_params(ba, gm, bt, bb):
    C = ba.shape[0]
    p = jnp.zeros((8, C), jnp.float32)
    return p.at[0].set(ba).at[1].set(gm).at[2].set(bt).at[3].set(bb)


def kernel(pos, normal, W1a, b1a, g1, be1, W1b, b1b, W2a, b2a, g2, be2, W2b, b2b,
           W3a, b3a, g3, be3, W3b, b3b):
    posq = jnp.concatenate(
        [pos, jnp.full((NPAD - N, 3), 1e4, jnp.float32)], axis=0)
    D2, seg, ck = _knn_stage_a(posq, posq.T)
    Dr = _sc_gather(D2, seg[:N].reshape(-1), K * N)
    nbr = _knn_stage_b(Dr.reshape(N, K * 128), ck[:N], N, P=400)
    nbr_t = nbr.T

    x0 = jnp.concatenate([pos, normal], axis=-1)

    idx_flat = nbr_t.reshape(-1)

    def layer(x, Wa, ba, gm, bt, Wb, bb):
        A, B = _node_ab(x, pos, Wa)
        C = Wa.shape[1]
        Cg = max(128, C)
        Ap = A if C == Cg else jnp.pad(A, ((0, 0), (0, Cg - C)))
        G = _sc_gather(Ap, idx_flat, K * N).reshape(K, N, Cg)
        return _conv_layer(G, B, Wb, _pack_params(ba, gm, bt, bb),
                           _group_avg_matrix(C), P=400)

    h1 = layer(x0, W1a, b1a, g1, be1, W1b, b1b)
    h2 = layer(h1, W2a, b2a, g2, be2, W2b, b2b)
    h3 = layer(h2, W3a, b3a, g3, be3, W3b, b3b)
    return (h1, h2, h3)


# speedup vs baseline: 1.0497x; 1.0497x over previous
import functools

import jax
import jax.numpy as jnp
from jax import lax
from jax.experimental import pallas as pl
from jax.experimental.pallas import tpu as pltpu
from jax.experimental.pallas import tpu_sc as plsc

N = 10000
K = 16
_EPS = 1e-5
NPAD = 10240
NCH = NPAD // 128

_SC_NC = 2
_SC_NS = 16
_SC_NW = _SC_NC * _SC_NS


def _sc_gather(table, idx, n_rows, c_out=None):
    T, C = table.shape
    assert C % 128 == 0
    c_out = C if c_out is None else c_out
    n_chunks = n_rows // 128
    cpw = (n_chunks + _SC_NW - 1) // _SC_NW
    cpw = ((cpw + 7) // 8) * 8
    n_pad = cpw * _SC_NW
    lsz = ((cpw + 8 + 7) // 8) * 8
    idx2 = jnp.zeros((n_pad + 32, 128), jnp.int32).at[:n_chunks].set(
        idx.reshape(n_chunks, 128))
    mesh = plsc.VectorSubcoreMesh(core_axis_name="c", subcore_axis_name="s")

    assert cpw % 4 == 0

    @functools.partial(
        pl.kernel,
        mesh=mesh,
        out_type=jax.ShapeDtypeStruct((n_pad * 128, c_out), jnp.float32),
        scratch_types=[
            pltpu.VMEM((lsz, 128), jnp.int32),
            pltpu.VMEM((128, C), jnp.float32),
            pltpu.VMEM((128, C), jnp.float32),
            pltpu.SemaphoreType.DMA,
            pltpu.SemaphoreType.DMA,
        ],
    )
    def gk(idx_hbm, table_hbm, out_hbm, idx_v, rows0, rows1, sem0, sem1):
        wid = lax.axis_index("s") * _SC_NC + lax.axis_index("c")
        base = wid * cpw
        ab = pl.multiple_of((base // 8) * 8, 8)
        off = base - ab
        pltpu.sync_copy(idx_hbm.at[pl.ds(ab, lsz)], idx_v)
        pltpu.async_copy(table_hbm.at[idx_v.at[off]], rows0, sem0)

        def store(rows, j):
            src = rows if c_out == C else rows.at[:, pl.ds(0, c_out)]
            pltpu.sync_copy(src, out_hbm.at[pl.ds((base + j) * 128, 128)])

        def body(h, carry):
            j = 2 * h
            pltpu.async_copy(table_hbm.at[idx_v.at[j + 1]], rows1, sem1)
            pltpu.make_async_copy(table_hbm.at[idx_v.at[j]], rows0, sem0).wait()
            store(rows0, j)

            @pl.when(j + 2 < cpw)
            def _():
                pltpu.async_copy(table_hbm.at[idx_v.at[j + 2]], rows0, sem0)

            pltpu.make_async_copy(table_hbm.at[idx_v.at[j + 1]], rows1, sem1).wait()
            store(rows1, j + 1)
            return carry

        lax.fori_loop(0, cpw // 2, body, 0)

    return gk(idx2, table)[:n_rows]


_QB = 256


def _knn_a_body(q_ref, pt_ref, d_ref, seg_ref, ck_ref):
    q = q_ref[...]
    pt = pt_ref[...]
    t = jnp.dot(q, pt, preferred_element_type=jnp.float32)
    sqc = jnp.sum(pt * pt, axis=0, keepdims=True)
    d = sqc - 2.0 * t
    mins = []
    for c in range(NCH):
        blk = d[:, c * 128:(c + 1) * 128]
        d_ref[pl.ds(c * _QB, _QB), :] = blk
        mins.append(jnp.min(blk, axis=1, keepdims=True))
    m = jnp.concatenate(mins, axis=1)
    iota = lax.broadcasted_iota(jnp.int32, (_QB, NCH), 1).astype(jnp.float32)
    qloc = lax.broadcasted_iota(jnp.int32, (_QB, 1), 0)
    cols = []
    for _ in range(K):
        rm = jnp.min(m, axis=1, keepdims=True)
        cand = jnp.where(m == rm, iota, jnp.float32(1e9))
        pc = jnp.min(cand, axis=1, keepdims=True)
        m = jnp.where(cand == pc, jnp.float32(jnp.inf), m)
        cols.append(pc.astype(jnp.int32))
    ck = jnp.concatenate(cols, axis=1)
    ck_ref[...] = ck
    seg_ref[...] = pl.program_id(0) * (NCH * _QB) + ck * _QB + qloc


def _knn_stage_a(posq, post):
    return pl.pallas_call(
        _knn_a_body,
        grid=(NPAD // _QB,),
        in_specs=[
            pl.BlockSpec((_QB, 3), lambda i: (i, 0)),
            pl.BlockSpec((3, NPAD), lambda i: (0, 0)),
        ],
        out_specs=[
            pl.BlockSpec((NCH * _QB, 128), lambda i: (i, 0)),
            pl.BlockSpec((_QB, K), lambda i: (i, 0)),
            pl.BlockSpec((_QB, K), lambda i: (i, 0)),
        ],
        out_shape=[
            jax.ShapeDtypeStruct((NPAD * NCH, 128), jnp.float32),
            jax.ShapeDtypeStruct((NPAD, K), jnp.int32),
            jax.ShapeDtypeStruct((NPAD, K), jnp.int32),
        ],
    )(posq, post)


def _knn_b_body(dr_ref, ck_ref, nbr_ref):
    P = dr_ref.shape[0]
    d = dr_ref[...]
    ck_f = ck_ref[...].astype(jnp.float32)
    iota2 = lax.broadcasted_iota(jnp.int32, (P, K * 128), 1).astype(jnp.float32)
    iota16 = lax.broadcasted_iota(jnp.int32, (P, K), 1).astype(jnp.float32)
    cols = []
    for _ in range(K):
        rm = jnp.min(d, axis=1, keepdims=True)
        cand = jnp.where(d == rm, iota2, jnp.float32(1e9))
        pc = jnp.min(cand, axis=1, keepdims=True)
        d = jnp.where(cand == pc, jnp.float32(jnp.inf), d)
        j2 = jnp.floor(pc * (1.0 / 128.0))
        lane = pc - j2 * 128.0
        sel = jnp.sum(jnp.where(iota16 == j2, ck_f, 0.0), axis=1, keepdims=True)
        cols.append((sel * 128.0 + lane).astype(jnp.int32))
    nbr_ref[...] = jnp.concatenate(cols, axis=1)


def _knn_stage_b(dr, ck, n, P=200):
    return pl.pallas_call(
        _knn_b_body,
        grid=(n // P,),
        in_specs=[
            pl.BlockSpec((P, K * 128), lambda i: (i, 0)),
            pl.BlockSpec((P, K), lambda i: (i, 0)),
        ],
        out_specs=pl.BlockSpec((P, K), lambda i: (i, 0)),
        out_shape=jax.ShapeDtypeStruct((n, K), jnp.int32),
    )(dr, ck)


def _ab_body(x_ref, p_ref, wt_ref, wb_ref, a_ref, b_ref):
    bvec = jnp.dot(p_ref[...], wb_ref[...], preferred_element_type=jnp.float32)
    a_ref[...] = jnp.dot(x_ref[...], wt_ref[...], preferred_element_type=jnp.float32) + bvec
    b_ref[...] = bvec


def _node_ab(x, pos, Wa):
    fin = x.shape[1]
    C = Wa.shape[1]
    wt = Wa[:fin]
    wb = Wa[fin:]
    return pl.pallas_call(
        _ab_body,
        out_shape=(
            jax.ShapeDtypeStruct((N, C), jnp.float32),
            jax.ShapeDtypeStruct((N, C), jnp.float32),
        ),
    )(x, pos, wt, wb)


def _conv_body(g_ref, b_ref, wb_ref, s_ref, prm_ref, o_ref):
    Kc, P, _ = g_ref.shape
    C = b_ref.shape[1]
    ba = prm_ref[0:1, :]
    gm = prm_ref[1:2, :]
    bt = prm_ref[2:3, :]
    bb = prm_ref[3:4, :]
    s = s_ref[...]
    h = (g_ref[..., :C] - b_ref[...][None]).reshape(Kc * P, C) + ba
    m = jnp.dot(h, s, preferred_element_type=jnp.float32)
    d = h - m
    v = jnp.dot(d * d, s, preferred_element_type=jnp.float32)
    hn = (d / jnp.sqrt(v + _EPS)) * gm + bt
    hr = jnp.maximum(hn, 0.0)
    z = jnp.dot(hr, wb_ref[...], preferred_element_type=jnp.float32)
    acc = jnp.max(z.reshape(Kc, P, C), axis=0)
    o_ref[...] = jnp.maximum(acc + bb, 0.0)


def _conv_layer(G, B, Wb, prm, S, P=200):
    n = B.shape[0]
    C = B.shape[1]
    Cg = G.shape[2]
    grid = (n // P,)
    return pl.pallas_call(
        _conv_body,
        grid=grid,
        in_specs=[
            pl.BlockSpec((K, P, Cg), lambda i: (0, i, 0)),
            pl.BlockSpec((P, C), lambda i: (i, 0)),
            pl.BlockSpec((C, C), lambda i: (0, 0)),
            pl.BlockSpec((C, C), lambda i: (0, 0)),
            pl.BlockSpec((8, C), lambda i: (0, 0)),
        ],
        out_specs=pl.BlockSpec((P, C), lambda i: (i, 0)),
        out_shape=jax.ShapeDtypeStruct((n, C), jnp.float32),
    )(G, B, Wb, S, prm)


def _group_avg_matrix(C):
    i = jnp.arange(C)
    return jnp.where((i[:, None] // 8) == (i[None, :] // 8), 1.0 / 8.0, 0.0).astype(jnp.float32)


def _pack_params(ba, gm, bt, bb):
    C = ba.shape[0]
    p = jnp.zeros((8, C), jnp.float32)
    return p.at[0].set(ba).at[1].set(gm).at[2].set(bt).at[3].set(bb)


def kernel(pos, normal, W1a, b1a, g1, be1, W1b, b1b, W2a, b2a, g2, be2, W2b, b2b,
           W3a, b3a, g3, be3, W3b, b3b):
    posq = jnp.concatenate(
        [pos, jnp.full((NPAD - N, 3), 1e4, jnp.float32)], axis=0)
    D2, seg, ck = _knn_stage_a(posq, posq.T)
    Dr = _sc_gather(D2, seg[:N].reshape(-1), K * N)
    nbr = _knn_stage_b(Dr.reshape(N, K * 128), ck[:N], N, P=400)
    nbr_t = nbr.T

    x0 = jnp.concatenate([pos, normal], axis=-1)

    idx_flat = nbr_t.reshape(-1)

    def layer(x, Wa, ba, gm, bt, Wb, bb):
        A, B = _node_ab(x, pos, Wa)
        C = Wa.shape[1]
        Cg = max(128, C)
        Ap = A if C == Cg else jnp.pad(A, ((0, 0), (0, Cg - C)))
        G = _sc_gather(Ap, idx_flat, K * N).reshape(K, N, Cg)
        return _conv_layer(G, B, Wb, _pack_params(ba, gm, bt, bb),
                           _group_avg_matrix(C), P=400)

    h1 = layer(x0, W1a, b1a, g1, be1, W1b, b1b)
    h2 = layer(h1, W2a, b2a, g2, be2, W2b, b2b)
    h3 = layer(h2, W3a, b3a, g3, be3, W3b, b3b)
    return (h1, h2, h3)
